# baseline (speedup 1.0000x reference)
import functools

import jax
import jax.numpy as jnp
from jax import lax
from jax.experimental import pallas as pl
from jax.experimental.pallas import tpu as pltpu
from jax.experimental.pallas import tpu_sc as plsc


def kernel(x, pos_emb_weight):
    Bx, Tx, Dx = x.shape
    NC, NS = 2, 16
    NW = NC * NS
    rows = Bx * Tx
    rpw = rows // NW
    R = 32
    n_chunks = rpw // R
    xf = x.reshape(rows, Dx)
    row_ids = jnp.arange(rows, dtype=jnp.int32)
    mesh = plsc.VectorSubcoreMesh(core_axis_name="c", subcore_axis_name="s")

    @functools.partial(
        pl.kernel,
        mesh=mesh,
        out_type=jax.ShapeDtypeStruct((rows, Dx), jnp.float32),
        scratch_types=[
            pltpu.VMEM((R, Dx), jnp.float32),
            pltpu.VMEM((R, Dx), jnp.float32),
        ],
    )
    def sc_add(x_hbm, pos_hbm, out_hbm, bufx, bufp):
        sid = lax.axis_index("s")
        wid = sid * NC + lax.axis_index("c")
        base = wid * rpw
        pos_base = lax.rem(base, Tx)

        def body(c, carry):
            lo = base + c * R
            plo = pos_base + c * R
            pltpu.sync_copy(x_hbm.at[pl.ds(lo, R)], bufx)
            pltpu.sync_copy(pos_hbm.at[pl.ds(plo, R)], bufp)

            def radd(r, _):
                for k in range(Dx // 16):
                    cs = pl.ds(k * 16, 16)
                    bufx[r, cs] = bufx[r, cs] + bufp[r, cs]
                return _

            lax.fori_loop(0, R, radd, 0)
            pltpu.sync_copy(bufx, out_hbm.at[pl.ds(lo, R)])
            return carry

        lax.fori_loop(0, n_chunks, body, 0)

    out = sc_add(xf, pos_emb_weight[:Tx])
    return out.reshape(Bx, Tx, Dx)


# speedup vs baseline: 1.0587x; 1.0587x over previous
import functools

import jax
import jax.numpy as jnp
from jax import lax
from jax.experimental import pallas as pl
from jax.experimental.pallas import tpu as pltpu
from jax.experimental.pallas import tpu_sc as plsc

_NC, _NS = 2, 16


def _sc_part(xb, pos):
    rows, Dx = xb.shape
    Tx = pos.shape[0]
    NW = _NC * _NS
    rpw = rows // NW
    R = 32
    n_chunks = rpw // R
    mesh = plsc.VectorSubcoreMesh(core_axis_name="c", subcore_axis_name="s")

    @functools.partial(
        pl.kernel,
        mesh=mesh,
        out_type=jax.ShapeDtypeStruct((rows, Dx), jnp.float32),
        scratch_types=[
            pltpu.VMEM((R, Dx), jnp.float32),
            pltpu.VMEM((R, Dx), jnp.float32),
        ],
    )
    def sc_add(x_hbm, pos_hbm, out_hbm, bufx, bufp):
        sid = lax.axis_index("s")
        wid = sid * _NC + lax.axis_index("c")
        base = wid * rpw
        pos_base = lax.rem(base, Tx)

        def body(c, carry):
            lo = base + c * R
            plo = pos_base + c * R
            pltpu.sync_copy(x_hbm.at[pl.ds(lo, R)], bufx)
            pltpu.sync_copy(pos_hbm.at[pl.ds(plo, R)], bufp)

            def radd(r, _):
                for k in range(Dx // 16):
                    cs = pl.ds(k * 16, 16)
                    bufx[r, cs] = bufx[r, cs] + bufp[r, cs]
                return _

            lax.fori_loop(0, R, radd, 0)
            pltpu.sync_copy(bufx, out_hbm.at[pl.ds(lo, R)])
            return carry

        lax.fori_loop(0, n_chunks, body, 0)

    return sc_add(xb, pos)


def _tc_add_kernel(x_ref, pos_ref, out_ref):
    out_ref[...] = x_ref[...] + pos_ref[...]


def _tc_part(xb, pos):
    rows, Dx = xb.shape
    Tx = pos.shape[0]
    nb = rows // Tx
    RB = 2048
    n_chunks = Tx // RB
    return pl.pallas_call(
        _tc_add_kernel,
        grid=(n_chunks, nb),
        in_specs=[
            pl.BlockSpec((RB, Dx), lambda p, b: (b * n_chunks + p, 0)),
            pl.BlockSpec((RB, Dx), lambda p, b: (p, 0)),
        ],
        out_specs=pl.BlockSpec((RB, Dx), lambda p, b: (b * n_chunks + p, 0)),
        out_shape=jax.ShapeDtypeStruct((rows, Dx), jnp.float32),
        compiler_params=pltpu.CompilerParams(
            dimension_semantics=("arbitrary", "arbitrary"),
        ),
    )(xb, pos)


def kernel(x, pos_emb_weight):
    Bx, Tx, Dx = x.shape
    pos = pos_emb_weight[:Tx]
    SB = 1
    xf = x.reshape(Bx * Tx, Dx)
    sc_out = _sc_part(xf[: SB * Tx], pos)
    tc_out = _tc_part(xf[SB * Tx :], pos)
    out = jnp.concatenate([sc_out, tc_out], axis=0)
    return out.reshape(Bx, Tx, Dx)


# speedup vs baseline: 3.2860x; 3.1037x over previous
import jax
import jax.numpy as jnp
from jax.experimental import pallas as pl
from jax.experimental.pallas import tpu as pltpu


def _add_kernel(x_ref, pos_ref, out_ref):
    out_ref[...] = x_ref[...] + pos_ref[...]


def kernel(x, pos_emb_weight):
    Bx, Tx, Dx = x.shape
    RB = 4096
    DB = 512
    n_chunks = Tx // RB
    n_d = Dx // DB
    xf = x.reshape(Bx * Tx, Dx)
    out = pl.pallas_call(
        _add_kernel,
        grid=(n_chunks, n_d, Bx),
        in_specs=[
            pl.BlockSpec((RB, DB), lambda p, d, b: (b * n_chunks + p, d)),
            pl.BlockSpec((RB, DB), lambda p, d, b: (p, d)),
        ],
        out_specs=pl.BlockSpec((RB, DB), lambda p, d, b: (b * n_chunks + p, d)),
        out_shape=jax.ShapeDtypeStruct((Bx * Tx, Dx), x.dtype),
        compiler_params=pltpu.CompilerParams(
            dimension_semantics=("arbitrary", "arbitrary", "arbitrary"),
        ),
    )(xf, pos_emb_weight[:Tx])
    return out.reshape(Bx, Tx, Dx)
